# baseline (speedup 1.0000x reference)
import functools
import math

import jax
import jax.numpy as jnp
from jax import lax
from jax.experimental import pallas as pl
from jax.experimental.pallas import tpu as pltpu
from jax.experimental.pallas import tpu_sc as plsc

N, C, H, W = 4, 256, 100, 352
HW = H * W
K = int(math.ceil(0.1 * HW))
BLK = 7040
NB = HW // BLK

NWORK = 32
CPW = (N * HW) // NWORK
NCHUNK16 = CPW // 16
CH = 64
IDXCAP = CPW + CH


def _scores_kernel(x_ref, w1_ref, b1_ref, w2_ref, b2_ref, dis_ref, s_ref,
                   z_ref):
    xm = x_ref[0]
    h = jax.lax.dot_general(xm, w1_ref[...], (((1,), (0,)), ((), ())))
    h = jnp.maximum(h + b1_ref[...], 0.0)
    logits = jax.lax.dot_general(w2_ref[...], h, (((0,), (1,)), ((), ())))
    score = jax.nn.sigmoid(logits + b2_ref[...])
    s_ref[0] = score * dis_ref[0]
    z_ref[0] = jnp.zeros((BLK, C), jnp.float32)


def _thresh_kernel(s_ref, t_ref):
    bits = jax.lax.bitcast_convert_type(s_ref[...], jnp.int32)
    lo0 = jnp.zeros((N, 1, 1), jnp.int32)
    hi0 = jnp.full((N, 1, 1), 0x7F800000, jnp.int32)

    def body(_, carry):
        lo, hi = carry
        mid = lo + (hi - lo) // 2
        cnt = jnp.sum((bits >= mid).astype(jnp.int32), axis=(1, 2),
                      keepdims=True)
        ge = cnt >= K
        return jnp.where(ge, mid, lo), jnp.where(ge, hi, mid)

    lo, _ = jax.lax.fori_loop(0, 31, body, (lo0, hi0))
    t_ref[...] = jax.lax.bitcast_convert_type(lo.reshape(N, 1), jnp.float32)


def _sc_scatter(x_hbm, s_hbm, t_hbm, out_hbm,
                s_v, t_v, idx_flat, sv_flat, idx_small, sv_small, rows_v,
                sem):
    wid = lax.axis_index("s") * 2 + lax.axis_index("c")
    base = wid * CPW
    agent = wid // (NWORK // N)

    pltpu.sync_copy(t_hbm, t_v)
    pltpu.sync_copy(s_hbm.at[pl.ds(base, CPW)], s_v)
    lanes = lax.iota(jnp.int32, 16)
    tv = t_v[...]
    a_spl = jnp.full((16,), agent, jnp.int32)
    t_spl = jnp.where(
        a_spl == 0, jnp.full((16,), tv[0], jnp.float32),
        jnp.where(a_spl == 1, jnp.full((16,), tv[1], jnp.float32),
                  jnp.where(a_spl == 2, jnp.full((16,), tv[2], jnp.float32),
                            jnp.full((16,), tv[3], jnp.float32))))

    def compact(j, off):
        sv = s_v[pl.ds(j * 16, 16)]
        m = sv >= t_spl
        ids = (base + j * 16) + lanes
        plsc.store_compressed(idx_flat.at[pl.ds(off, 16)], ids, mask=m)
        plsc.store_compressed(sv_flat.at[pl.ds(off, 16)], sv, mask=m)
        cnt = plsc.all_reduce_population_count(m)[0]
        return off + cnt

    cnt = lax.fori_loop(0, NCHUNK16, compact, jnp.int32(0))

    lastpos = jnp.maximum(cnt - 1, 0)
    last_i = jnp.full((16,), idx_flat[pl.ds(lastpos, 16)][0], jnp.int32)
    last_s = jnp.full((16,), sv_flat[pl.ds(lastpos, 16)][0], jnp.float32)
    for u in range(CH // 16):
        idx_flat[pl.ds(cnt + u * 16, 16)] = last_i
        sv_flat[pl.ds(cnt + u * 16, 16)] = last_s

    nch = (cnt + (CH - 1)) // CH

    def chunk(j, _):
        for u in range(CH // 16):
            idx_small[0, pl.ds(u * 16, 16)] = idx_flat[pl.ds(j * CH + u * 16, 16)]
            sv_small[pl.ds(u * 16, 16)] = sv_flat[pl.ds(j * CH + u * 16, 16)]
        pltpu.async_copy(x_hbm.at[idx_small.at[0]], rows_v, sem).wait()

        def rowloop(r, _):
            spl = jnp.full((16,), sv_small[pl.ds(r, 16)][0], jnp.float32)
            for u in range(C // 16):
                rows_v[r, pl.ds(u * 16, 16)] = rows_v[r, pl.ds(u * 16, 16)] * spl
            return 0

        lax.fori_loop(0, CH, rowloop, 0)
        pltpu.async_copy(rows_v, out_hbm.at[idx_small.at[0]], sem).wait()
        return 0

    lax.fori_loop(0, nch, chunk, 0)


def kernel(spatial_features_2d, dis_priority, w1, b1, w2, b2):
    xt = spatial_features_2d.transpose(0, 2, 3, 1).reshape(N, HW, C)
    dis = dis_priority.reshape(N, 1, HW)
    b1r = b1.reshape(1, C)
    b2r = b2.reshape(1, 1)

    scores, canvas = pl.pallas_call(
        _scores_kernel,
        grid=(N, NB),
        in_specs=[
            pl.BlockSpec((1, BLK, C), lambda n, b: (n, b, 0)),
            pl.BlockSpec((C, C), lambda n, b: (0, 0)),
            pl.BlockSpec((1, C), lambda n, b: (0, 0)),
            pl.BlockSpec((C, 1), lambda n, b: (0, 0)),
            pl.BlockSpec((1, 1), lambda n, b: (0, 0)),
            pl.BlockSpec((1, 1, BLK), lambda n, b: (n, 0, b)),
        ],
        out_specs=[
            pl.BlockSpec((1, 1, BLK), lambda n, b: (n, 0, b)),
            pl.BlockSpec((1, BLK, C), lambda n, b: (n, b, 0)),
        ],
        out_shape=[
            jax.ShapeDtypeStruct((N, 1, HW), jnp.float32),
            jax.ShapeDtypeStruct((N, HW, C), jnp.float32),
        ],
    )(xt, w1, b1r, w2, b2r, dis)

    thresh = pl.pallas_call(
        _thresh_kernel,
        out_shape=jax.ShapeDtypeStruct((N, 1), jnp.float32),
    )(scores)

    s_flat = scores.reshape(N * HW)
    t16 = jnp.pad(thresh.reshape(N), (0, 16 - N))
    x2 = xt.reshape(N * HW, C)
    canvas2 = canvas.reshape(N * HW, C)

    mesh = plsc.VectorSubcoreMesh(core_axis_name="c", subcore_axis_name="s")
    canvas_ref = jax.new_ref(canvas2)
    pl.kernel(
        _sc_scatter,
        mesh=mesh,
        out_type=(),
        compiler_params=pltpu.CompilerParams(needs_layout_passes=False),
        scratch_types=[
            pltpu.VMEM((CPW,), jnp.float32),
            pltpu.VMEM((16,), jnp.float32),
            pltpu.VMEM((IDXCAP,), jnp.int32),
            pltpu.VMEM((IDXCAP,), jnp.float32),
            pltpu.VMEM((1, CH), jnp.int32),
            pltpu.VMEM((CH + 16,), jnp.float32),
            pltpu.VMEM((CH, C), jnp.float32),
            pltpu.SemaphoreType.DMA,
        ],
    )(x2, s_flat, t16, canvas_ref)

    out2 = canvas_ref[...]
    return out2.reshape(N, H, W, C).transpose(0, 3, 1, 2)


# speedup vs baseline: 1.0170x; 1.0170x over previous
import functools
import math

import jax
import jax.numpy as jnp
from jax import lax
from jax.experimental import pallas as pl
from jax.experimental.pallas import tpu as pltpu
from jax.experimental.pallas import tpu_sc as plsc

N, C, H, W = 4, 256, 100, 352
HW = H * W
K = int(math.ceil(0.1 * HW))
BLK = 7040
NB = HW // BLK

NWORK = 32
CPW = (N * HW) // NWORK
NCHUNK16 = CPW // 16
CH = 64
IDXCAP = CPW + 2 * CH


def _scores_kernel(x_ref, w1_ref, b1_ref, w2_ref, b2_ref, dis_ref, s_ref,
                   z_ref):
    xm = x_ref[0]
    h = jax.lax.dot_general(xm, w1_ref[...], (((1,), (0,)), ((), ())))
    h = jnp.maximum(h + b1_ref[...], 0.0)
    logits = jax.lax.dot_general(w2_ref[...], h, (((0,), (1,)), ((), ())))
    score = jax.nn.sigmoid(logits + b2_ref[...])
    s_ref[0] = score * dis_ref[0]
    z_ref[0] = jnp.zeros((BLK, C), jnp.float32)


def _thresh_kernel(s_ref, t_ref):
    bits = jax.lax.bitcast_convert_type(s_ref[...], jnp.int32)
    lo0 = jnp.zeros((N, 1, 1), jnp.int32)
    hi0 = jnp.full((N, 1, 1), 0x7F800000, jnp.int32)

    def body(_, carry):
        lo, hi = carry
        mid = lo + (hi - lo) // 2
        cnt = jnp.sum((bits >= mid).astype(jnp.int32), axis=(1, 2),
                      keepdims=True)
        ge = cnt >= K
        return jnp.where(ge, mid, lo), jnp.where(ge, hi, mid)

    lo, _ = jax.lax.fori_loop(0, 31, body, (lo0, hi0))
    t_ref[...] = jax.lax.bitcast_convert_type(lo.reshape(N, 1), jnp.float32)


def _sc_scatter(x_hbm, s_hbm, t_hbm, out_hbm,
                s_v, t_v, idx_flat, sv_flat,
                idx_sa, sv_sa, rows_a, idx_sb, sv_sb, rows_b,
                sem_ga, sem_gb, sem_sa, sem_sb):
    wid = lax.axis_index("s") * 2 + lax.axis_index("c")
    base = wid * CPW
    agent = wid // (NWORK // N)

    pltpu.sync_copy(t_hbm, t_v)
    pltpu.sync_copy(s_hbm.at[pl.ds(base, CPW)], s_v)
    lanes = lax.iota(jnp.int32, 16)
    tv = t_v[...]
    a_spl = jnp.full((16,), agent, jnp.int32)
    t_spl = jnp.where(
        a_spl == 0, jnp.full((16,), tv[0], jnp.float32),
        jnp.where(a_spl == 1, jnp.full((16,), tv[1], jnp.float32),
                  jnp.where(a_spl == 2, jnp.full((16,), tv[2], jnp.float32),
                            jnp.full((16,), tv[3], jnp.float32))))

    def compact(j, off):
        sv = s_v[pl.ds(j * 16, 16)]
        m = sv >= t_spl
        ids = (base + j * 16) + lanes
        plsc.store_compressed(idx_flat.at[pl.ds(off, 16)], ids, mask=m)
        plsc.store_compressed(sv_flat.at[pl.ds(off, 16)], sv, mask=m)
        cnt = plsc.all_reduce_population_count(m)[0]
        return off + cnt

    cnt = lax.fori_loop(0, NCHUNK16, compact, jnp.int32(0))

    lastpos = jnp.maximum(cnt - 1, 0)
    last_i = jnp.full((16,), idx_flat[pl.ds(lastpos, 16)][0], jnp.int32)
    last_s = jnp.full((16,), sv_flat[pl.ds(lastpos, 16)][0], jnp.float32)
    for u in range(2 * CH // 16):
        idx_flat[pl.ds(cnt + u * 16, 16)] = last_i
        sv_flat[pl.ds(cnt + u * 16, 16)] = last_s

    npair = (cnt + (2 * CH - 1)) // (2 * CH)

    def stage(j, idx_s, sv_s):
        for u in range(CH // 16):
            idx_s[0, pl.ds(u * 16, 16)] = idx_flat[pl.ds(j * CH + u * 16, 16)]
            sv_s[pl.ds(u * 16, 16)] = sv_flat[pl.ds(j * CH + u * 16, 16)]

    def scale(rows, sv_s):
        def rowloop(r, _):
            spl = jnp.full((16,), sv_s[pl.ds(r, 16)][0], jnp.float32)
            for u in range(C // 16):
                rows[r, pl.ds(u * 16, 16)] = rows[r, pl.ds(u * 16, 16)] * spl
            return 0
        lax.fori_loop(0, CH, rowloop, 0)

    def pair(jj, _):
        j0 = 2 * jj
        stage(j0, idx_sa, sv_sa)
        ga = pltpu.async_copy(x_hbm.at[idx_sa.at[0]], rows_a, sem_ga)
        stage(j0 + 1, idx_sb, sv_sb)
        gb = pltpu.async_copy(x_hbm.at[idx_sb.at[0]], rows_b, sem_gb)
        ga.wait()
        scale(rows_a, sv_sa)
        sa = pltpu.async_copy(rows_a, out_hbm.at[idx_sa.at[0]], sem_sa)
        gb.wait()
        scale(rows_b, sv_sb)
        sb = pltpu.async_copy(rows_b, out_hbm.at[idx_sb.at[0]], sem_sb)
        sa.wait()
        sb.wait()
        return 0

    lax.fori_loop(0, npair, pair, 0)


def kernel(spatial_features_2d, dis_priority, w1, b1, w2, b2):
    xt = spatial_features_2d.transpose(0, 2, 3, 1).reshape(N, HW, C)
    dis = dis_priority.reshape(N, 1, HW)
    b1r = b1.reshape(1, C)
    b2r = b2.reshape(1, 1)

    scores, canvas = pl.pallas_call(
        _scores_kernel,
        grid=(N, NB),
        in_specs=[
            pl.BlockSpec((1, BLK, C), lambda n, b: (n, b, 0)),
            pl.BlockSpec((C, C), lambda n, b: (0, 0)),
            pl.BlockSpec((1, C), lambda n, b: (0, 0)),
            pl.BlockSpec((C, 1), lambda n, b: (0, 0)),
            pl.BlockSpec((1, 1), lambda n, b: (0, 0)),
            pl.BlockSpec((1, 1, BLK), lambda n, b: (n, 0, b)),
        ],
        out_specs=[
            pl.BlockSpec((1, 1, BLK), lambda n, b: (n, 0, b)),
            pl.BlockSpec((1, BLK, C), lambda n, b: (n, b, 0)),
        ],
        out_shape=[
            jax.ShapeDtypeStruct((N, 1, HW), jnp.float32),
            jax.ShapeDtypeStruct((N, HW, C), jnp.float32),
        ],
    )(xt, w1, b1r, w2, b2r, dis)

    thresh = pl.pallas_call(
        _thresh_kernel,
        out_shape=jax.ShapeDtypeStruct((N, 1), jnp.float32),
    )(scores)

    s_flat = scores.reshape(N * HW)
    t16 = jnp.pad(thresh.reshape(N), (0, 16 - N))
    x2 = xt.reshape(N * HW, C)
    canvas2 = canvas.reshape(N * HW, C)

    mesh = plsc.VectorSubcoreMesh(core_axis_name="c", subcore_axis_name="s")
    canvas_ref = jax.new_ref(canvas2)
    pl.kernel(
        _sc_scatter,
        mesh=mesh,
        out_type=(),
        compiler_params=pltpu.CompilerParams(needs_layout_passes=False),
        scratch_types=[
            pltpu.VMEM((CPW,), jnp.float32),
            pltpu.VMEM((16,), jnp.float32),
            pltpu.VMEM((IDXCAP,), jnp.int32),
            pltpu.VMEM((IDXCAP,), jnp.float32),
            pltpu.VMEM((1, CH), jnp.int32),
            pltpu.VMEM((CH + 16,), jnp.float32),
            pltpu.VMEM((CH, C), jnp.float32),
            pltpu.VMEM((1, CH), jnp.int32),
            pltpu.VMEM((CH + 16,), jnp.float32),
            pltpu.VMEM((CH, C), jnp.float32),
            pltpu.SemaphoreType.DMA,
            pltpu.SemaphoreType.DMA,
            pltpu.SemaphoreType.DMA,
            pltpu.SemaphoreType.DMA,
        ],
    )(x2, s_flat, t16, canvas_ref)

    out2 = canvas_ref[...]
    return out2.reshape(N, H, W, C).transpose(0, 3, 1, 2)
